# Initial kernel scaffold; baseline (speedup 1.0000x reference)
import jax
import jax.numpy as jnp
from jax import lax
from jax.experimental import pallas as pl
from jax.experimental.pallas import tpu as pltpu
from jax.experimental.pallas import tpu_sc as plsc

HIDDEN = 64
BATCH = 16384
SEQ = 50
NC, NS, LANES = 2, 16, 16
NW = NC * NS
ROWS_PER_W = BATCH // NW
CHUNK = 16
IDX_PER_GATHER = 100
NSUB = CHUNK * SEQ // IDX_PER_GATHER
NCHUNKS = ROWS_PER_W // CHUNK
NVH = HIDDEN // LANES


def _sc_body(x2d_hbm, table_hbm, out_hbm, idx_v, rows_v, out_v, sem):
    wid = lax.axis_index("s") * NC + lax.axis_index("c")
    wbase = wid * ROWS_PER_W
    inv = jnp.float32(1.0 / SEQ)

    def chunk_body(i, carry):
        rowbase = wbase + i * CHUNK
        idxrow = rowbase // 2
        pltpu.sync_copy(x2d_hbm.at[pl.ds(idxrow, NSUB)], idx_v)
        cps = [
            pltpu.async_copy(
                table_hbm.at[idx_v.at[j]],
                rows_v.at[pl.ds(j * IDX_PER_GATHER, IDX_PER_GATHER)],
                sem,
            )
            for j in range(NSUB)
        ]
        for cp in cps:
            cp.wait()
        for c in range(CHUNK):
            def s_body(s, accs):
                r = c * SEQ + s
                return tuple(
                    accs[h] + rows_v[r, pl.ds(h * LANES, LANES)]
                    for h in range(NVH)
                )
            z = jnp.zeros((LANES,), jnp.float32)
            accs = lax.fori_loop(0, SEQ, s_body, (z,) * NVH)
            for h in range(NVH):
                out_v[c, pl.ds(h * LANES, LANES)] = accs[h] * inv
        pltpu.sync_copy(out_v, out_hbm.at[pl.ds(rowbase, CHUNK)])
        return carry

    lax.fori_loop(0, NCHUNKS, chunk_body, 0)


def kernel(x, table):
    x2d = x.astype(jnp.int32).reshape(BATCH * SEQ // IDX_PER_GATHER,
                                      IDX_PER_GATHER)
    k = pl.kernel(
        _sc_body,
        out_type=jax.ShapeDtypeStruct((BATCH, HIDDEN), jnp.float32),
        mesh=plsc.VectorSubcoreMesh(core_axis_name="c", subcore_axis_name="s"),
        scratch_types=[
            pltpu.VMEM((NSUB, IDX_PER_GATHER), jnp.int32),
            pltpu.VMEM((CHUNK * SEQ, HIDDEN), jnp.float32),
            pltpu.VMEM((CHUNK, HIDDEN), jnp.float32),
            pltpu.SemaphoreType.DMA,
        ],
    )
    return k(x2d, table)


# speedup vs baseline: 2.4423x; 2.4423x over previous
import jax
import jax.numpy as jnp
from jax import lax
from jax.experimental import pallas as pl
from jax.experimental.pallas import tpu as pltpu
from jax.experimental.pallas import tpu_sc as plsc

HIDDEN = 64
BATCH = 16384
SEQ = 50
NC, NS, LANES = 2, 16, 16
NW = NC * NS
ROWS_PER_W = BATCH // NW
CHUNK = 16
IDX_PER_GATHER = 100
NSUB = CHUNK * SEQ // IDX_PER_GATHER
NCHUNKS = ROWS_PER_W // CHUNK
NVH = HIDDEN // LANES


def _sc_body(x2d_hbm, table_hbm, out_hbm, idx_v, rows_v, out_v, sem):
    wid = lax.axis_index("s") * NC + lax.axis_index("c")
    wbase = wid * ROWS_PER_W
    inv = jnp.float32(1.0 / SEQ)

    def chunk_body(i, carry):
        rowbase = wbase + i * CHUNK
        idxrow = pl.multiple_of(rowbase // 2, 8)
        pltpu.sync_copy(x2d_hbm.at[pl.ds(idxrow, NSUB)], idx_v)
        cps = [
            pltpu.async_copy(
                table_hbm.at[idx_v.at[j]],
                rows_v.at[pl.ds(j * IDX_PER_GATHER, IDX_PER_GATHER)],
                sem,
            )
            for j in range(NSUB)
        ]
        for cp in cps:
            cp.wait()
        for c in range(CHUNK):
            def s_body(s, accs):
                r = c * SEQ + s
                return tuple(
                    accs[h] + rows_v[r, pl.ds(h * LANES, LANES)]
                    for h in range(NVH)
                )
            z = jnp.zeros((LANES,), jnp.float32)
            accs = lax.fori_loop(0, SEQ, s_body, (z,) * NVH)
            for h in range(NVH):
                out_v[c, pl.ds(h * LANES, LANES)] = accs[h] * inv
        pltpu.sync_copy(out_v, out_hbm.at[pl.ds(rowbase, CHUNK)])
        return carry

    lax.fori_loop(0, NCHUNKS, chunk_body, 0)


def kernel(x, table):
    x2d = x.astype(jnp.int32).reshape(BATCH * SEQ // IDX_PER_GATHER,
                                      IDX_PER_GATHER)
    k = pl.kernel(
        _sc_body,
        out_type=jax.ShapeDtypeStruct((BATCH, HIDDEN), jnp.float32),
        mesh=plsc.VectorSubcoreMesh(core_axis_name="c", subcore_axis_name="s"),
        compiler_params=pltpu.CompilerParams(use_tc_tiling_on_sc=False),
        scratch_types=[
            pltpu.VMEM((NSUB, IDX_PER_GATHER), jnp.int32),
            pltpu.VMEM((CHUNK * SEQ, HIDDEN), jnp.float32),
            pltpu.VMEM((CHUNK, HIDDEN), jnp.float32),
            pltpu.SemaphoreType.DMA,
        ],
    )
    return k(x2d, table)


# speedup vs baseline: 2.8648x; 1.1730x over previous
import jax
import jax.numpy as jnp
from jax import lax
from jax.experimental import pallas as pl
from jax.experimental.pallas import tpu as pltpu
from jax.experimental.pallas import tpu_sc as plsc

HIDDEN = 64
BATCH = 16384
SEQ = 50
NC, NS, LANES = 2, 16, 16
NW = NC * NS
ROWS_PER_W = BATCH // NW
CHUNK = 128
NCHUNKS = ROWS_PER_W // CHUNK
NVH = HIDDEN // LANES


def _sc_body(xt_hbm, table_hbm, out_hbm, idx_v, acc_v, sem):
    wid = lax.axis_index("s") * NC + lax.axis_index("c")
    wbase = wid * ROWS_PER_W
    inv = jnp.float32(1.0 / SEQ)

    def chunk_body(i, carry):
        rowbase = pl.multiple_of(wbase + i * CHUNK, CHUNK)
        pltpu.sync_copy(xt_hbm.at[:, pl.ds(rowbase, CHUNK)], idx_v)
        pltpu.async_copy(table_hbm.at[idx_v.at[0]], acc_v, sem).wait()
        cps = [
            pltpu.async_copy(table_hbm.at[idx_v.at[s]], acc_v, sem, add=True)
            for s in range(1, SEQ)
        ]
        for cp in cps:
            cp.wait()

        def scale_body(c, carry2):
            for h in range(NVH):
                sl = pl.ds(h * LANES, LANES)
                acc_v[c, sl] = acc_v[c, sl] * inv
            return carry2

        lax.fori_loop(0, CHUNK, scale_body, 0)
        pltpu.sync_copy(acc_v, out_hbm.at[pl.ds(rowbase, CHUNK)])
        return carry

    lax.fori_loop(0, NCHUNKS, chunk_body, 0)


def kernel(x, table):
    xt = x.astype(jnp.int32).T
    k = pl.kernel(
        _sc_body,
        out_type=jax.ShapeDtypeStruct((BATCH, HIDDEN), jnp.float32),
        mesh=plsc.VectorSubcoreMesh(core_axis_name="c", subcore_axis_name="s"),
        compiler_params=pltpu.CompilerParams(use_tc_tiling_on_sc=False),
        scratch_types=[
            pltpu.VMEM((SEQ, CHUNK), jnp.int32),
            pltpu.VMEM((CHUNK, HIDDEN), jnp.float32),
            pltpu.SemaphoreType.DMA,
        ],
    )
    return k(xt, table)
